# Initial kernel scaffold; baseline (speedup 1.0000x reference)
import jax
import jax.numpy as jnp
from jax.experimental import pallas as pl


def kernel(x, mask_d, weight):
    raise NotImplementedError("write your pallas kernel here")


# speedup vs baseline: 2.0186x; 2.0186x over previous
import functools

import jax
import jax.numpy as jnp
from jax import lax
from jax.experimental import pallas as pl
from jax.experimental.pallas import tpu as pltpu
from jax.experimental.pallas import tpu_sc as plsc

_NUM_ITEM = 100000
_HIDDEN = 128
_BATCH = 4096
_FIELDS = 26

_B = _BATCH * _FIELDS
_NC, _NS, _L = 2, 16, 16
_NW = _NC * _NS
_BPW = _B // _NW
_CH = 128
_NCHUNK = _BPW // _CH

_mesh = plsc.VectorSubcoreMesh(core_axis_name="c", subcore_axis_name="s")


@functools.partial(
    pl.kernel,
    mesh=_mesh,
    out_type=jax.ShapeDtypeStruct((_B, _HIDDEN), jnp.float32),
    scratch_types=[
        pltpu.VMEM((_BPW,), jnp.int32),
        pltpu.VMEM((_CH,), jnp.int32),
        pltpu.VMEM((_CH, _HIDDEN), jnp.float32),
        pltpu.SemaphoreType.DMA,
    ],
)
def _masked_lookup(x_hbm, mask_hbm, w_hbm, out_hbm, idx_v, mv_v, rows_v, sem):
    wid = lax.axis_index("s") * _NC + lax.axis_index("c")
    base = wid * _BPW
    pltpu.sync_copy(x_hbm.at[pl.ds(base, _BPW)], idx_v)

    def chunk_body(c, carry):
        cidx = idx_v.at[pl.ds(c * _CH, _CH)]
        pltpu.async_copy(mask_hbm.at[cidx], mv_v, sem).wait()
        pltpu.async_copy(w_hbm.at[cidx], rows_v, sem).wait()

        def group_body(g, gcarry):
            mvec = mv_v[pl.ds(g * _L, _L)]
            for r in range(_L):
                m = mvec[r]
                row = g * _L + r
                for j in range(_HIDDEN // _L):
                    h = lax.broadcasted_iota(jnp.int32, (_L,), 0) + (j * _L)
                    v = rows_v[row, pl.ds(j * _L, _L)]
                    rows_v[row, pl.ds(j * _L, _L)] = jnp.where(h <= m, v, 0.0)
            return gcarry

        lax.fori_loop(0, _CH // _L, group_body, 0)
        pltpu.sync_copy(rows_v, out_hbm.at[pl.ds(base + c * _CH, _CH)])
        return carry

    lax.fori_loop(0, _NCHUNK, chunk_body, 0)


def kernel(x, mask_d, weight):
    xf = x.reshape(_B).astype(jnp.int32)
    out = _masked_lookup(xf, mask_d.astype(jnp.int32), weight)
    return out.reshape(_BATCH, _FIELDS, _HIDDEN)


# speedup vs baseline: 3.1742x; 1.5725x over previous
import functools

import jax
import jax.numpy as jnp
from jax import lax
from jax.experimental import pallas as pl
from jax.experimental.pallas import tpu as pltpu
from jax.experimental.pallas import tpu_sc as plsc

_NUM_ITEM = 100000
_HIDDEN = 128
_BATCH = 4096
_FIELDS = 26

_B = _BATCH * _FIELDS
_NC, _NS, _L = 2, 16, 16
_NW = _NC * _NS
_BPW = _BATCH // _NW
_CB = 8
_CH = _CB * _FIELDS
_NCHUNK = _BPW // _CB
_HG = _CH // 2

_mesh = plsc.VectorSubcoreMesh(core_axis_name="c", subcore_axis_name="s")


@functools.partial(
    pl.kernel,
    mesh=_mesh,
    out_type=jax.ShapeDtypeStruct((_BATCH, _FIELDS, _HIDDEN), jnp.float32),
    scratch_types=[
        pltpu.VMEM((_BPW * _FIELDS,), jnp.int32),
        pltpu.VMEM((_CH,), jnp.int32),
        pltpu.VMEM((_CH, _HIDDEN), jnp.float32),
        pltpu.SemaphoreType.DMA,
    ],
)
def _masked_lookup(x_hbm, mask_hbm, w_hbm, out_hbm, idx_v, mv_v, rows_v, sem):
    wid = lax.axis_index("s") * _NC + lax.axis_index("c")
    pltpu.sync_copy(x_hbm.at[pl.ds(wid * _BPW * _FIELDS, _BPW * _FIELDS)], idx_v)

    def chunk_body(c, carry):
        handles = []
        for h in range(2):
            cidx = idx_v.at[pl.ds(c * _CH + h * _HG, _HG)]
            handles.append(pltpu.async_copy(
                mask_hbm.at[cidx], mv_v.at[pl.ds(h * _HG, _HG)], sem))
            handles.append(pltpu.async_copy(
                w_hbm.at[cidx], rows_v.at[pl.ds(h * _HG, _HG)], sem))
        for hd in handles:
            hd.wait()

        def group_body(g, gcarry):
            mvec = mv_v[pl.ds(g * _L, _L)]
            for r in range(_L):
                m = mvec[r]
                row = g * _L + r
                for j in range(_HIDDEN // _L):
                    h = lax.broadcasted_iota(jnp.int32, (_L,), 0) + (j * _L)
                    v = rows_v[row, pl.ds(j * _L, _L)]
                    rows_v[row, pl.ds(j * _L, _L)] = jnp.where(h <= m, v, 0.0)
            return gcarry

        lax.fori_loop(0, _CH // _L, group_body, 0)

        for b in range(_CB):
            pltpu.sync_copy(rows_v.at[pl.ds(b * _FIELDS, _FIELDS)],
                            out_hbm.at[wid * _BPW + c * _CB + b])
        return carry

    lax.fori_loop(0, _NCHUNK, chunk_body, 0)


def kernel(x, mask_d, weight):
    xf = x.reshape(_B).astype(jnp.int32)
    return _masked_lookup(xf, mask_d.astype(jnp.int32), weight)


# speedup vs baseline: 4.0084x; 1.2628x over previous
import functools

import jax
import jax.numpy as jnp
from jax import lax
from jax.experimental import pallas as pl
from jax.experimental.pallas import tpu as pltpu
from jax.experimental.pallas import tpu_sc as plsc

_NUM_ITEM = 100000
_HIDDEN = 128
_BATCH = 4096
_FIELDS = 26

_B = _BATCH * _FIELDS
_NC, _NS, _L = 2, 16, 16
_NW = _NC * _NS
_BPW = _BATCH // _NW
_CB = 8
_CH = _CB * _FIELDS
_NCHUNK = _BPW // _CB
_HG = _CH // 2
_NBUF = 4

_mesh = plsc.VectorSubcoreMesh(core_axis_name="c", subcore_axis_name="s")


@functools.partial(
    pl.kernel,
    mesh=_mesh,
    out_type=jax.ShapeDtypeStruct((_BATCH, _FIELDS, _HIDDEN), jnp.float32),
    scratch_types=[
        pltpu.VMEM((_BPW * _FIELDS,), jnp.int32),
        pltpu.VMEM((_NBUF * _CH,), jnp.int32),
        pltpu.VMEM((_NBUF * _CH, _HIDDEN), jnp.float32),
    ]
    + [pltpu.SemaphoreType.DMA] * (2 * _NBUF),
)
def _masked_lookup(x_hbm, mask_hbm, w_hbm, out_hbm, idx_v, mv_v, rows_v, *sems):
    g_sems, o_sems = sems[:_NBUF], sems[_NBUF:]
    wid = lax.axis_index("s") * _NC + lax.axis_index("c")
    pltpu.sync_copy(x_hbm.at[pl.ds(wid * _BPW * _FIELDS, _BPW * _FIELDS)], idx_v)

    def gather_copies(c, p, fn):
        for h in range(2):
            cidx = idx_v.at[pl.ds(c * _CH + h * _HG, _HG)]
            fn(mask_hbm.at[cidx], mv_v.at[pl.ds(p * _CH + h * _HG, _HG)], g_sems[p])
            fn(w_hbm.at[cidx], rows_v.at[pl.ds(p * _CH + h * _HG, _HG)], g_sems[p])

    def out_copies(c, p, fn):
        for b in range(_CB):
            fn(rows_v.at[pl.ds(p * _CH + b * _FIELDS, _FIELDS)],
               out_hbm.at[wid * _BPW + c * _CB + b], o_sems[p])

    def fire(src, dst, sem):
        pltpu.async_copy(src, dst, sem)

    def drain(src, dst, sem):
        pltpu.make_async_copy(src, dst, sem).wait()

    def compute(p):
        def group_body(g, gcarry):
            mvec = mv_v[pl.ds(p * _CH + g * _L, _L)]
            for r in range(_L):
                m = mvec[r]
                row = p * _CH + g * _L + r
                for j in range(_HIDDEN // _L):
                    h = lax.broadcasted_iota(jnp.int32, (_L,), 0) + (j * _L)
                    v = rows_v[row, pl.ds(j * _L, _L)]
                    rows_v[row, pl.ds(j * _L, _L)] = jnp.where(h <= m, v, 0.0)
            return gcarry

        lax.fori_loop(0, _CH // _L, group_body, 0)

    gather_copies(0, 0, fire)

    def iter_body(i, carry):
        for p in range(_NBUF):
            c = _NBUF * i + p
            p1 = (p + 1) % _NBUF
            gather_copies(c, p, drain)
            if p == _NBUF - 1:
                out_copies(c - 3, p1, drain)

                @pl.when(i < _NCHUNK // _NBUF - 1)
                def _():
                    gather_copies(c + 1, p1, fire)
            else:
                @pl.when(i > 0)
                def _():
                    out_copies(c - 3, p1, drain)
                gather_copies(c + 1, p1, fire)
            compute(p)
            out_copies(c, p, fire)
        return carry

    lax.fori_loop(0, _NCHUNK // _NBUF, iter_body, 0)
    for c in (_NCHUNK - 3, _NCHUNK - 2, _NCHUNK - 1):
        out_copies(c, c % _NBUF, drain)


def kernel(x, mask_d, weight):
    xf = x.reshape(_B).astype(jnp.int32)
    return _masked_lookup(xf, mask_d.astype(jnp.int32), weight)
